# Initial kernel scaffold; baseline (speedup 1.0000x reference)
import jax
import jax.numpy as jnp
from jax.experimental import pallas as pl


def kernel(x, edge_index, edge_weight, W1, b1, W2, b2, W3, b3):
    raise NotImplementedError("write your pallas kernel here")


# speedup vs baseline: 9.0169x; 9.0169x over previous
import dataclasses
import functools

import jax
import jax.numpy as jnp
from jax import lax
from jax.experimental import pallas as pl
from jax.experimental.pallas import tpu as pltpu
from jax.experimental.pallas import tpu_sc as plsc

NC = 2
NS = 16
NW = NC * NS
LANES = 16

V_PAD = 10240
ROWS_PER_TILE = V_PAD // NS
ZB = 32
EDGE_WIN = 200

_mesh = plsc.VectorSubcoreMesh(core_axis_name="c", subcore_axis_name="s")

_sc_params = pltpu.CompilerParams()
if "needs_layout_passes" in pltpu.CompilerParams.__dataclass_fields__:
    _sc_params = dataclasses.replace(_sc_params, needs_layout_passes=False)


def _deg_body(col_hbm, ew_hbm, out_hbm, acc_sh, colv, eww, stg):
    cid = lax.axis_index("c")
    sid = lax.axis_index("s")
    nseg = ROWS_PER_TILE

    @pl.loop(0, nseg // LANES)
    def _(i):
        stg[pl.ds(i * LANES, LANES)] = jnp.zeros((LANES,), jnp.float32)

    pltpu.sync_copy(stg, acc_sh.at[pl.ds(sid * nseg, nseg)])
    plsc.subcore_barrier()

    e_total = col_hbm.shape[0]
    per_tile = e_total // NW
    ebase = (cid * NS + sid) * per_tile
    win = per_tile // 5

    @pl.loop(0, 5)
    def _(w):
        b = ebase + w * win
        pltpu.sync_copy(col_hbm.at[pl.ds(b, win)], colv)
        pltpu.sync_copy(ew_hbm.at[pl.ds(b, win)], eww)
        pltpu.sync_copy(eww, acc_sh.at[colv], add=True)

    plsc.subcore_barrier()
    pltpu.sync_copy(acc_sh.at[pl.ds(sid * nseg, nseg)], stg)
    pltpu.sync_copy(stg, out_hbm.at[cid, pl.ds(sid * nseg, nseg)])


def _deg_call(col, ew):
    e_total = col.shape[0]
    per_tile = e_total // NW
    win = per_tile // 5
    kern = pl.kernel(
        _deg_body,
        out_type=jax.ShapeDtypeStruct((NC, V_PAD), jnp.float32),
        mesh=_mesh,
        compiler_params=_sc_params,
        scratch_types=[
            pltpu.VMEM_SHARED((V_PAD,), jnp.float32),
            pltpu.VMEM((win,), jnp.int32),
            pltpu.VMEM((win,), jnp.float32),
            pltpu.VMEM((ROWS_PER_TILE,), jnp.float32),
        ],
    )
    return kern(col, ew)


def _norm_body(row_hbm, col_hbm, ew_hbm, dis_hbm, out_hbm,
               disv, roww, colw, eww, normw):
    cid = lax.axis_index("c")
    sid = lax.axis_index("s")
    pltpu.sync_copy(dis_hbm, disv)

    e_total = row_hbm.shape[0]
    per_tile = e_total // NW
    ebase = (cid * NS + sid) * per_tile
    win = roww.shape[0]

    @pl.loop(0, per_tile // win)
    def _(w):
        b = ebase + w * win
        pltpu.sync_copy(row_hbm.at[pl.ds(b, win)], roww)
        pltpu.sync_copy(col_hbm.at[pl.ds(b, win)], colw)
        pltpu.sync_copy(ew_hbm.at[pl.ds(b, win)], eww)

        @pl.loop(0, win // LANES)
        def _(j):
            sl = pl.ds(j * LANES, LANES)
            a = plsc.load_gather(disv, [roww[sl]])
            c = plsc.load_gather(disv, [colw[sl]])
            normw[sl] = a * c * eww[sl]

        pltpu.sync_copy(normw, out_hbm.at[pl.ds(b, win)])


def _norm_call(row, col, ew, dis_flat):
    e_total = row.shape[0]
    win = EDGE_WIN
    kern = pl.kernel(
        _norm_body,
        out_type=jax.ShapeDtypeStruct((e_total,), jnp.float32),
        mesh=_mesh,
        compiler_params=_sc_params,
        scratch_types=[
            pltpu.VMEM((V_PAD,), jnp.float32),
            pltpu.VMEM((win,), jnp.int32),
            pltpu.VMEM((win,), jnp.int32),
            pltpu.VMEM((win,), jnp.float32),
            pltpu.VMEM((win,), jnp.float32),
        ],
    )
    return kern(row, col, ew, dis_flat)


def _prop_body(h_hbm, row_hbm, col_hbm, norm_hbm, out_hbm,
               acc_sh, gbuf, zbuf, rowv, colv, normw, sem):
    cid = lax.axis_index("c")
    sid = lax.axis_index("s")

    @pl.loop(0, ZB)
    def _(i):
        for cchunk in range(8):
            zbuf[i, pl.ds(cchunk * LANES, LANES)] = (
                jnp.zeros((LANES,), jnp.float32))

    @pl.loop(0, ROWS_PER_TILE // ZB)
    def _(i):
        pltpu.sync_copy(zbuf, acc_sh.at[pl.ds(sid * ROWS_PER_TILE + i * ZB, ZB)])

    plsc.subcore_barrier()

    e_total = row_hbm.shape[0]
    per_tile = e_total // NW
    ebase = (cid * NS + sid) * per_tile
    win = rowv.shape[0]

    @pl.loop(0, per_tile // win)
    def _(w):
        b = ebase + w * win
        pltpu.sync_copy(row_hbm.at[pl.ds(b, win)], rowv)
        pltpu.sync_copy(col_hbm.at[pl.ds(b, win)], colv)
        pltpu.sync_copy(norm_hbm.at[pl.ds(b, win)], normw)
        pltpu.async_copy(h_hbm.at[rowv], gbuf, sem).wait()

        @pl.loop(0, win)
        def _(e):
            ns = plsc.load_gather(normw, [jnp.full((LANES,), e, jnp.int32)])
            for cchunk in range(8):
                slc = (e, pl.ds(cchunk * LANES, LANES))
                gbuf[slc] = gbuf[slc] * ns

        pltpu.sync_copy(gbuf, acc_sh.at[colv], add=True)

    plsc.subcore_barrier()

    @pl.loop(0, ROWS_PER_TILE // ZB)
    def _(i):
        r = sid * ROWS_PER_TILE + i * ZB
        pltpu.sync_copy(acc_sh.at[pl.ds(r, ZB)], zbuf)
        pltpu.sync_copy(zbuf, out_hbm.at[cid, pl.ds(r, ZB)])


def _prop_call(h, row, col, norm):
    kern = pl.kernel(
        _prop_body,
        out_type=jax.ShapeDtypeStruct((NC, V_PAD, 128), jnp.float32),
        mesh=_mesh,
        compiler_params=_sc_params,
        scratch_types=[
            pltpu.VMEM_SHARED((V_PAD, 128), jnp.float32),
            pltpu.VMEM((EDGE_WIN, 128), jnp.float32),
            pltpu.VMEM((ZB, 128), jnp.float32),
            pltpu.VMEM((EDGE_WIN,), jnp.int32),
            pltpu.VMEM((EDGE_WIN,), jnp.int32),
            pltpu.VMEM((EDGE_WIN,), jnp.float32),
            pltpu.SemaphoreType.DMA,
        ],
    )
    return kern(h, row, col, norm)


def _dis_kernel(deg_ref, dis_ref, dis2_ref):
    d = deg_ref[0] + deg_ref[1] + 1.0
    r = lax.rsqrt(d)
    dis_ref[...] = r
    dis2_ref[...] = r * r


def _dis_call(deg_parts):
    dp = deg_parts.reshape(NC, V_PAD // 128, 128)
    return pl.pallas_call(
        _dis_kernel,
        out_shape=(jax.ShapeDtypeStruct((V_PAD // 128, 128), jnp.float32),
                   jax.ShapeDtypeStruct((V_PAD // 128, 128), jnp.float32)),
    )(dp)


def _mm_kernel(p_ref, h_ref, d2_ref, w_ref, b_ref, o_ref):
    xblk = p_ref[0] + p_ref[1] + d2_ref[...] * h_ref[...]
    o_ref[...] = lax.dot_general(
        xblk, w_ref[...], (((1,), (0,)), ((), ())),
        preferred_element_type=jnp.float32,
        precision=lax.Precision.HIGHEST) + b_ref[...]


def _mm_call(p, h, dis2b, w, b):
    blk = 1024
    grid = (V_PAD // blk,)
    return pl.pallas_call(
        _mm_kernel,
        grid=grid,
        in_specs=[
            pl.BlockSpec((NC, blk, 128), lambda i: (0, i, 0)),
            pl.BlockSpec((blk, 128), lambda i: (i, 0)),
            pl.BlockSpec((blk, 128), lambda i: (i, 0)),
            pl.BlockSpec((128, 128), lambda i: (0, 0)),
            pl.BlockSpec((1, 128), lambda i: (0, 0)),
        ],
        out_specs=pl.BlockSpec((blk, 128), lambda i: (i, 0)),
        out_shape=jax.ShapeDtypeStruct((V_PAD, 128), jnp.float32),
    )(p, h, dis2b, w, b)


def _relu_kernel(p_ref, h_ref, d2_ref, b_ref, o_ref):
    o_ref[...] = jnp.maximum(
        p_ref[0] + p_ref[1] + d2_ref[...] * h_ref[...] + b_ref[...], 0.0)


def _relu_call(p, h, dis2b, b):
    blk = 1024
    grid = (V_PAD // blk,)
    return pl.pallas_call(
        _relu_kernel,
        grid=grid,
        in_specs=[
            pl.BlockSpec((NC, blk, 128), lambda i: (0, i, 0)),
            pl.BlockSpec((blk, 128), lambda i: (i, 0)),
            pl.BlockSpec((blk, 128), lambda i: (i, 0)),
            pl.BlockSpec((1, 128), lambda i: (0, 0)),
        ],
        out_specs=pl.BlockSpec((blk, 128), lambda i: (i, 0)),
        out_shape=jax.ShapeDtypeStruct((V_PAD, 128), jnp.float32),
    )(p, h, dis2b, b)


def _fold_kernel(a_ref, w3_ref, o_ref):
    o_ref[...] = lax.dot_general(
        a_ref[...], w3_ref[...], (((1,), (0,)), ((), ())),
        preferred_element_type=jnp.float32,
        precision=lax.Precision.HIGHEST)


def _fold_call(w2b, w3):
    return pl.pallas_call(
        _fold_kernel,
        out_shape=jax.ShapeDtypeStruct((136, 128), jnp.float32),
    )(w2b, w3)


def kernel(x, edge_index, edge_weight, W1, b1, W2, b2, W3, b3):
    B, F_in, T, V = x.shape
    S = B * T
    row = edge_index[0]
    col = edge_index[1]

    deg_parts = _deg_call(col, edge_weight)
    dis, dis2 = _dis_call(deg_parts)
    dis_flat = dis.reshape(V_PAD)
    dis2b = jnp.broadcast_to(dis2.reshape(V_PAD, 1), (V_PAD, 128))
    norm = _norm_call(row, col, edge_weight, dis_flat)

    w2b = jnp.concatenate(
        [W2, b2[None, :], jnp.zeros((7, W2.shape[1]), jnp.float32)], axis=0)
    fold = _fold_call(w2b, W3)
    W23 = fold[:128]
    b23 = fold[128:129]

    b1r = b1.reshape(1, 128)
    b3r = b3.reshape(1, 128)

    xs = jnp.transpose(x, (0, 2, 3, 1)).reshape(S, V, F_in)
    xs = jnp.pad(xs, ((0, 0), (0, V_PAD - V), (0, 0)))

    outs = []
    for s in range(S):
        h0 = xs[s]
        p1 = _prop_call(h0, row, col, norm)
        h1 = _mm_call(p1, h0, dis2b, W1, b1r)
        p2 = _prop_call(h1, row, col, norm)
        y = _mm_call(p2, h1, dis2b, W23, b23)
        p3 = _prop_call(y, row, col, norm)
        outs.append(_relu_call(p3, y, dis2b, b3r))

    res = jnp.stack(outs)[:, :V, :]
    res = res.reshape(B, T, V, 128)
    return jnp.transpose(res, (0, 3, 1, 2))
